# Initial kernel scaffold; baseline (speedup 1.0000x reference)
import jax
import jax.numpy as jnp
from jax.experimental import pallas as pl


def kernel(x, conn0, op0, conn1, op1, conn2, op2, conn3, op3):
    raise NotImplementedError("write your pallas kernel here")


# speedup vs baseline: 451.2759x; 451.2759x over previous
import functools

import jax
import jax.numpy as jnp
from jax import lax
from jax.experimental import pallas as pl
from jax.experimental.pallas import tpu as pltpu
from jax.experimental.pallas import tpu_sc as plsc

NC = 2
NS = 16
NW = NC * NS
L = 16
CPN = 128
UNROLL = 16


def _make_layer(in_f, out_f, chunk=64):
  npw = out_f // NW
  assert out_f % (NW * chunk) == 0 and chunk % L == 0
  mesh = plsc.VectorSubcoreMesh(core_axis_name="c", subcore_axis_name="s")

  @functools.partial(
      pl.kernel,
      mesh=mesh,
      out_type=jax.ShapeDtypeStruct((out_f,), jnp.float32),
      compiler_params=pltpu.CompilerParams(needs_layout_passes=False),
      scratch_types=[
          pltpu.VMEM((in_f,), jnp.float32),
          pltpu.VMEM((chunk * CPN,), jnp.int32),
          pltpu.VMEM((npw,), jnp.int32),
          pltpu.VMEM((npw,), jnp.float32),
      ],
  )
  def layer(x_hbm, conn_hbm, op_hbm, out_hbm, x_v, conn_v, op_v, out_v):
    wid = lax.axis_index("s") * NC + lax.axis_index("c")
    row0 = wid * npw
    pltpu.sync_copy(x_hbm, x_v)
    pltpu.sync_copy(op_hbm.at[pl.ds(row0, npw)], op_v)
    lanes = lax.iota(jnp.int32, L)

    def chunk_body(ci, _):
      pltpu.sync_copy(
          conn_hbm.at[pl.ds((row0 + ci * chunk) * CPN, chunk * CPN)], conn_v)

      def block_body(nb, _):
        rowbase = (lanes + nb * L) * CPN
        mn0 = jnp.full((L,), jnp.inf, jnp.float32)
        mx0 = jnp.full((L,), -jnp.inf, jnp.float32)

        def j_body(jo, carry):
          mn, mx = carry
          jbase = jo * UNROLL
          for jj in range(UNROLL):
            cidx = plsc.load_gather(conn_v, [rowbase + (jbase + jj)])
            v = plsc.load_gather(x_v, [cidx])
            mn = jnp.minimum(mn, v)
            mx = jnp.maximum(mx, v)
          return mn, mx

        mn, mx = lax.fori_loop(0, CPN // UNROLL, j_body, (mn0, mx0))
        o = ci * chunk + nb * L
        ops = op_v[pl.ds(o, L)]
        out_v[pl.ds(o, L)] = jnp.where(ops == 1, mx, mn)
        return 0

      lax.fori_loop(0, chunk // L, block_body, 0)
      return 0

    lax.fori_loop(0, npw // chunk, chunk_body, 0)
    pltpu.sync_copy(out_v, out_hbm.at[pl.ds(row0, npw)])

  return layer


_IN_F = 65536
_HID = [16384, 8192, 16384]
_SIZES = [_IN_F] + _HID + [_IN_F]
_LAYERS = [_make_layer(_SIZES[i], _SIZES[i + 1]) for i in range(4)]


def kernel(x, conn0, op0, conn1, op1, conn2, op2, conn3, op3):
  h0 = _LAYERS[0](x, conn0.reshape(-1), op0)
  h1 = _LAYERS[1](h0, conn1.reshape(-1), op1)
  h2 = _LAYERS[2](h1, conn2.reshape(-1), op2)
  h3 = _LAYERS[3](h2, conn3.reshape(-1), op3)
  return h3, jnp.concatenate([h0, h1, h2], axis=0)


# speedup vs baseline: 452.7217x; 1.0032x over previous
import functools

import jax
import jax.numpy as jnp
from jax import lax
from jax.experimental import pallas as pl
from jax.experimental.pallas import tpu as pltpu
from jax.experimental.pallas import tpu_sc as plsc

NC = 2
NS = 16
NW = NC * NS
L = 16
CPN = 128
UNROLL = 16


def _make_layer(in_f, out_f, chunk=64):
  npw = out_f // NW
  assert out_f % (NW * chunk) == 0 and chunk % L == 0
  mesh = plsc.VectorSubcoreMesh(core_axis_name="c", subcore_axis_name="s")

  @functools.partial(
      pl.kernel,
      mesh=mesh,
      out_type=jax.ShapeDtypeStruct((out_f,), jnp.float32),
      compiler_params=pltpu.CompilerParams(needs_layout_passes=False),
      scratch_types=[
          pltpu.VMEM((in_f,), jnp.float32),
          pltpu.VMEM((chunk * CPN,), jnp.int32),
          pltpu.VMEM((npw,), jnp.int32),
          pltpu.VMEM((npw,), jnp.float32),
      ],
  )
  def layer(x_hbm, conn_hbm, op_hbm, out_hbm, x_v, conn_v, op_v, out_v):
    wid = lax.axis_index("s") * NC + lax.axis_index("c")
    row0 = wid * npw
    pltpu.sync_copy(x_hbm, x_v)
    pltpu.sync_copy(op_hbm.at[pl.ds(row0, npw)], op_v)
    lanes = lax.iota(jnp.int32, L)

    def chunk_body(ci, _):
      pltpu.sync_copy(
          conn_hbm.at[pl.ds((row0 + ci * chunk) * CPN, chunk * CPN)], conn_v)

      def block_body(nb, _):
        rowbase = (lanes + nb * L) * CPN
        mn0 = jnp.full((L,), jnp.inf, jnp.float32)
        mx0 = jnp.full((L,), -jnp.inf, jnp.float32)

        def j_body(jo, carry):
          mn, mx, idx = carry
          cs = []
          for jj in range(UNROLL):
            cs.append(plsc.load_gather(conn_v, [idx]))
            idx = idx + 1
          vs = [plsc.load_gather(x_v, [c]) for c in cs]
          mns, mxs = list(vs), list(vs)
          while len(mns) > 1:
            mns = [jnp.minimum(a, b) for a, b in zip(mns[::2], mns[1::2])]
            mxs = [jnp.maximum(a, b) for a, b in zip(mxs[::2], mxs[1::2])]
          return jnp.minimum(mn, mns[0]), jnp.maximum(mx, mxs[0]), idx

        mn, mx, _ = lax.fori_loop(0, CPN // UNROLL, j_body,
                                  (mn0, mx0, rowbase))
        o = ci * chunk + nb * L
        ops = op_v[pl.ds(o, L)]
        out_v[pl.ds(o, L)] = jnp.where(ops == 1, mx, mn)
        return 0

      lax.fori_loop(0, chunk // L, block_body, 0)
      return 0

    lax.fori_loop(0, npw // chunk, chunk_body, 0)
    pltpu.sync_copy(out_v, out_hbm.at[pl.ds(row0, npw)])

  return layer


_IN_F = 65536
_HID = [16384, 8192, 16384]
_SIZES = [_IN_F] + _HID + [_IN_F]
_LAYERS = [_make_layer(_SIZES[i], _SIZES[i + 1]) for i in range(4)]


def kernel(x, conn0, op0, conn1, op1, conn2, op2, conn3, op3):
  h0 = _LAYERS[0](x, conn0.reshape(-1), op0)
  h1 = _LAYERS[1](h0, conn1.reshape(-1), op1)
  h2 = _LAYERS[2](h1, conn2.reshape(-1), op2)
  h3 = _LAYERS[3](h2, conn3.reshape(-1), op3)
  return h3, jnp.concatenate([h0, h1, h2], axis=0)


# speedup vs baseline: 784.4148x; 1.7327x over previous
import functools

import jax
import jax.numpy as jnp
from jax import lax
from jax.experimental import pallas as pl
from jax.experimental.pallas import tpu as pltpu
from jax.experimental.pallas import tpu_sc as plsc

NC = 2
NS = 16
NW = NC * NS
L = 16
CPN = 128
UNROLL = 16


def _make_layer(in_f, out_f, chunk=64):
  npw = out_f // NW
  assert out_f % (NW * chunk) == 0 and chunk % L == 0
  mesh = plsc.VectorSubcoreMesh(core_axis_name="c", subcore_axis_name="s")

  @functools.partial(
      pl.kernel,
      mesh=mesh,
      out_type=jax.ShapeDtypeStruct((out_f,), jnp.float32),
      compiler_params=pltpu.CompilerParams(needs_layout_passes=False),
      scratch_types=[
          pltpu.VMEM((in_f,), jnp.float32),
          pltpu.VMEM((chunk * CPN,), jnp.int32),
          pltpu.VMEM((npw,), jnp.int32),
          pltpu.VMEM((npw,), jnp.float32),
          pltpu.VMEM((L * 17,), jnp.float32),
          pltpu.VMEM((L * 17,), jnp.float32),
      ],
  )
  def layer(x_hbm, conn_hbm, op_hbm, out_hbm,
            x_v, conn_v, op_v, out_v, mn_buf, mx_buf):
    wid = lax.axis_index("s") * NC + lax.axis_index("c")
    row0 = wid * npw
    pltpu.sync_copy(x_hbm, x_v)
    pltpu.sync_copy(op_hbm.at[pl.ds(row0, npw)], op_v)
    lanes17 = lax.iota(jnp.int32, L) * 17

    def chunk_body(ci, _):
      pltpu.sync_copy(
          conn_hbm.at[pl.ds((row0 + ci * chunk) * CPN, chunk * CPN)], conn_v)

      def block_body(nb, _):
        base = nb * (L * CPN)
        for n in range(L):
          vs = []
          for jo in range(CPN // L):
            c = conn_v[pl.ds(base + n * CPN + jo * L, L)]
            vs.append(plsc.load_gather(x_v, [c]))
          mns, mxs = list(vs), list(vs)
          while len(mns) > 1:
            mns = [jnp.minimum(a, b) for a, b in zip(mns[::2], mns[1::2])]
            mxs = [jnp.maximum(a, b) for a, b in zip(mxs[::2], mxs[1::2])]
          mn_buf[pl.ds(n * 17, L)] = mns[0]
          mx_buf[pl.ds(n * 17, L)] = mxs[0]
        mns = [plsc.load_gather(mn_buf, [lanes17 + k]) for k in range(L)]
        mxs = [plsc.load_gather(mx_buf, [lanes17 + k]) for k in range(L)]
        while len(mns) > 1:
          mns = [jnp.minimum(a, b) for a, b in zip(mns[::2], mns[1::2])]
          mxs = [jnp.maximum(a, b) for a, b in zip(mxs[::2], mxs[1::2])]
        o = ci * chunk + nb * L
        ops = op_v[pl.ds(o, L)]
        out_v[pl.ds(o, L)] = jnp.where(ops == 1, mxs[0], mns[0])
        return 0

      lax.fori_loop(0, chunk // L, block_body, 0)
      return 0

    lax.fori_loop(0, npw // chunk, chunk_body, 0)
    pltpu.sync_copy(out_v, out_hbm.at[pl.ds(row0, npw)])

  return layer


_IN_F = 65536
_HID = [16384, 8192, 16384]
_SIZES = [_IN_F] + _HID + [_IN_F]
_LAYERS = [_make_layer(_SIZES[i], _SIZES[i + 1]) for i in range(4)]


def kernel(x, conn0, op0, conn1, op1, conn2, op2, conn3, op3):
  h0 = _LAYERS[0](x, conn0.reshape(-1), op0)
  h1 = _LAYERS[1](h0, conn1.reshape(-1), op1)
  h2 = _LAYERS[2](h1, conn2.reshape(-1), op2)
  h3 = _LAYERS[3](h2, conn3.reshape(-1), op3)
  return h3, jnp.concatenate([h0, h1, h2], axis=0)


# speedup vs baseline: 1059.7461x; 1.3510x over previous
import functools

import jax
import jax.numpy as jnp
from jax import lax
from jax.experimental import pallas as pl
from jax.experimental.pallas import tpu as pltpu
from jax.experimental.pallas import tpu_sc as plsc

NC = 2
NS = 16
NW = NC * NS
L = 16
CPN = 128


def _make_layer(in_f, out_f, chunk=128):
  npw = out_f // NW
  nchunks = npw // chunk
  assert out_f % (NW * chunk) == 0 and chunk % L == 0 and nchunks % 2 == 0
  mesh = plsc.VectorSubcoreMesh(core_axis_name="c", subcore_axis_name="s")

  @functools.partial(
      pl.kernel,
      mesh=mesh,
      out_type=jax.ShapeDtypeStruct((out_f,), jnp.float32),
      compiler_params=pltpu.CompilerParams(needs_layout_passes=False),
      scratch_types=[
          pltpu.VMEM((in_f,), jnp.float32),
          pltpu.VMEM((chunk * CPN,), jnp.int32),
          pltpu.VMEM((chunk * CPN,), jnp.int32),
          pltpu.VMEM((npw,), jnp.int32),
          pltpu.VMEM((npw,), jnp.float32),
          pltpu.VMEM((L * 17,), jnp.float32),
          pltpu.VMEM((L * 17,), jnp.float32),
          pltpu.SemaphoreType.DMA,
          pltpu.SemaphoreType.DMA,
      ],
  )
  def layer(x_hbm, conn_hbm, op_hbm, out_hbm,
            x_v, conn_a, conn_b, op_v, out_v, mn_buf, mx_buf, sem_a, sem_b):
    wid = lax.axis_index("s") * NC + lax.axis_index("c")
    row0 = wid * npw
    lanes17 = lax.iota(jnp.int32, L) * 17

    def conn_slice(c):
      return conn_hbm.at[pl.ds((row0 + c * chunk) * CPN, chunk * CPN)]

    pltpu.async_copy(conn_slice(0), conn_a, sem_a)
    pltpu.sync_copy(x_hbm, x_v)
    pltpu.sync_copy(op_hbm.at[pl.ds(row0, npw)], op_v)

    def do_chunk(conn_v, ci):
      def block_body(nb, _):
        base = nb * (L * CPN)
        for n in range(L):
          vs = []
          for jo in range(CPN // L):
            c = conn_v[pl.ds(base + n * CPN + jo * L, L)]
            vs.append(plsc.load_gather(x_v, [c]))
          mns, mxs = list(vs), list(vs)
          while len(mns) > 1:
            mns = [jnp.minimum(a, b) for a, b in zip(mns[::2], mns[1::2])]
            mxs = [jnp.maximum(a, b) for a, b in zip(mxs[::2], mxs[1::2])]
          mn_buf[pl.ds(n * 17, L)] = mns[0]
          mx_buf[pl.ds(n * 17, L)] = mxs[0]
        mns = [plsc.load_gather(mn_buf, [lanes17 + k]) for k in range(L)]
        mxs = [plsc.load_gather(mx_buf, [lanes17 + k]) for k in range(L)]
        while len(mns) > 1:
          mns = [jnp.minimum(a, b) for a, b in zip(mns[::2], mns[1::2])]
          mxs = [jnp.maximum(a, b) for a, b in zip(mxs[::2], mxs[1::2])]
        o = ci * chunk + nb * L
        ops = op_v[pl.ds(o, L)]
        out_v[pl.ds(o, L)] = jnp.where(ops == 1, mxs[0], mns[0])
        return 0

      lax.fori_loop(0, chunk // L, block_body, 0)

    def pair_body(p, _):
      c0 = 2 * p
      pltpu.async_copy(conn_slice(c0 + 1), conn_b, sem_b)
      pltpu.make_async_copy(conn_slice(c0), conn_a, sem_a).wait()
      do_chunk(conn_a, c0)

      @pl.when(c0 + 2 < nchunks)
      def _():
        pltpu.async_copy(conn_slice(c0 + 2), conn_a, sem_a)

      pltpu.make_async_copy(conn_slice(c0 + 1), conn_b, sem_b).wait()
      do_chunk(conn_b, c0 + 1)
      return 0

    lax.fori_loop(0, nchunks // 2, pair_body, 0)
    pltpu.sync_copy(out_v, out_hbm.at[pl.ds(row0, npw)])

  return layer


_IN_F = 65536
_HID = [16384, 8192, 16384]
_SIZES = [_IN_F] + _HID + [_IN_F]
_LAYERS = [_make_layer(_SIZES[i], _SIZES[i + 1]) for i in range(4)]


def kernel(x, conn0, op0, conn1, op1, conn2, op2, conn3, op3):
  h0 = _LAYERS[0](x, conn0.reshape(-1), op0)
  h1 = _LAYERS[1](h0, conn1.reshape(-1), op1)
  h2 = _LAYERS[2](h1, conn2.reshape(-1), op2)
  h3 = _LAYERS[3](h2, conn3.reshape(-1), op3)
  return h3, jnp.concatenate([h0, h1, h2], axis=0)
